# baseline (device time: 19154 ns/iter reference)
import jax
import jax.numpy as jnp
from jax import lax
from jax.experimental import pallas as pl
from jax.experimental.pallas import tpu as pltpu

N_DEV = 32
N_BLK = 8
BIG = 1e9


def _local_reduce(x):
    m_per, n = x.shape
    blk = m_per // N_BLK

    def body(x_ref, out_ref):
        k = pl.program_id(0)
        my_pos = lax.axis_index("i")

        xv = x_ref[:, :]
        bmax = jnp.max(xv, axis=0)
        rows = lax.broadcasted_iota(jnp.int32, (blk, n), 0)
        bidx = jnp.min(
            jnp.where(xv == bmax[None, :], rows, jnp.int32(blk)), axis=0
        )
        gidx = (my_pos * m_per + k * blk + bidx).astype(jnp.float32)

        @pl.when(k == 0)
        def _():
            out_ref[0, :] = bmax
            out_ref[1, :] = gidx

        @pl.when(k > 0)
        def _():
            v_old = out_ref[0, :]
            take = bmax > v_old
            out_ref[0, :] = jnp.where(take, bmax, v_old)
            out_ref[1, :] = jnp.where(take, gidx, out_ref[1, :])

    return pl.pallas_call(
        body,
        grid=(N_BLK,),
        out_shape=jax.ShapeDtypeStruct((2, n), jnp.float32),
        in_specs=[
            pl.BlockSpec((blk, n), lambda k: (k, 0), memory_space=pltpu.VMEM)
        ],
        out_specs=pl.BlockSpec((2, n), lambda k: (0, 0), memory_space=pltpu.VMEM),
    )(x)


def _all_to_all_argmax(local):
    _, n = local.shape

    def body(lb_ref, out_ref, recv_buf, send_sems, recv_sems):
        my_pos = lax.axis_index("i")

        barrier_sem = pltpu.get_barrier_semaphore()
        for j in range(N_DEV - 1):
            t = (my_pos + 1 + j) % N_DEV
            pl.semaphore_signal(
                barrier_sem, inc=1,
                device_id=(t,), device_id_type=pl.DeviceIdType.MESH,
            )
        pl.semaphore_wait(barrier_sem, N_DEV - 1)

        sends = []
        for j in range(N_DEV - 1):
            t = (my_pos + 1 + j) % N_DEV
            rdma = pltpu.make_async_remote_copy(
                src_ref=lb_ref,
                dst_ref=recv_buf.at[my_pos],
                send_sem=send_sems.at[j],
                recv_sem=recv_sems.at[my_pos],
                device_id=(t,),
                device_id_type=pl.DeviceIdType.MESH,
            )
            rdma.start()
            sends.append(rdma)

        for j in range(N_DEV - 1):
            s = (my_pos + 1 + j) % N_DEV
            recv = pltpu.make_async_remote_copy(
                src_ref=lb_ref,
                dst_ref=recv_buf.at[s],
                send_sem=send_sems.at[j],
                recv_sem=recv_sems.at[s],
                device_id=(s,),
                device_id_type=pl.DeviceIdType.MESH,
            )
            recv.wait_recv()
        for rdma in sends:
            rdma.wait_send()

        v = recv_buf[:, 0, :]
        i = recv_buf[:, 1, :]
        slot = lax.broadcasted_iota(jnp.int32, (N_DEV, n), 0)
        mine = slot == my_pos
        v = jnp.where(mine, jnp.float32(-jnp.inf), v)
        i = jnp.where(mine, BIG, i)
        v_l = lb_ref[0, :]
        i_l = lb_ref[1, :]
        vmax = jnp.maximum(jnp.max(v, axis=0), v_l)
        cand_r = jnp.min(jnp.where(v == vmax[None, :], i, BIG), axis=0)
        cand_l = jnp.where(v_l == vmax, i_l, BIG)
        out_ref[0, :] = vmax
        out_ref[1, :] = jnp.minimum(cand_r, cand_l)

    return pl.pallas_call(
        body,
        out_shape=jax.ShapeDtypeStruct((2, n), jnp.float32),
        in_specs=[pl.BlockSpec(memory_space=pltpu.VMEM)],
        out_specs=pl.BlockSpec(memory_space=pltpu.VMEM),
        scratch_shapes=[
            pltpu.VMEM((N_DEV, 2, n), jnp.float32),
            pltpu.SemaphoreType.DMA((N_DEV - 1,)),
            pltpu.SemaphoreType.DMA((N_DEV,)),
        ],
        compiler_params=pltpu.CompilerParams(collective_id=0),
    )(local)


def kernel(x):
    return _all_to_all_argmax(_local_reduce(x))


# device time: 17144 ns/iter; 1.1172x vs baseline; 1.1172x over previous
import jax
import jax.numpy as jnp
from jax import lax
from jax.experimental import pallas as pl
from jax.experimental.pallas import tpu as pltpu

N_DEV = 32
BIG = 1e9


def kernel(x):
    m_per, n = x.shape

    def body(x_ref, out_ref, local_ref, recv_buf, send_sems, recv_sems):
        my_pos = lax.axis_index("i")

        barrier_sem = pltpu.get_barrier_semaphore()
        for j in range(N_DEV - 1):
            t = (my_pos + 1 + j) % N_DEV
            pl.semaphore_signal(
                barrier_sem, inc=1,
                device_id=(t,), device_id_type=pl.DeviceIdType.MESH,
            )

        xv = x_ref[:, :]
        vmax_l = jnp.max(xv, axis=0)
        rows = lax.broadcasted_iota(jnp.int32, (m_per, n), 0)
        lidx = jnp.min(
            jnp.where(xv == vmax_l[None, :], rows, jnp.int32(m_per)), axis=0
        )
        gidx_l = (my_pos * m_per + lidx).astype(jnp.float32)
        local_ref[0, :] = vmax_l
        local_ref[1, :] = gidx_l

        pl.semaphore_wait(barrier_sem, N_DEV - 1)

        sends = []
        for j in range(N_DEV - 1):
            t = (my_pos + 1 + j) % N_DEV
            rdma = pltpu.make_async_remote_copy(
                src_ref=local_ref,
                dst_ref=recv_buf.at[my_pos],
                send_sem=send_sems.at[j],
                recv_sem=recv_sems.at[my_pos],
                device_id=(t,),
                device_id_type=pl.DeviceIdType.MESH,
            )
            rdma.start()
            sends.append(rdma)

        for j in range(N_DEV - 1):
            s = (my_pos + 1 + j) % N_DEV
            recv = pltpu.make_async_remote_copy(
                src_ref=local_ref,
                dst_ref=recv_buf.at[s],
                send_sem=send_sems.at[j],
                recv_sem=recv_sems.at[s],
                device_id=(s,),
                device_id_type=pl.DeviceIdType.MESH,
            )
            recv.wait_recv()
        for rdma in sends:
            rdma.wait_send()

        v = recv_buf[:, 0, :]
        i = recv_buf[:, 1, :]
        slot = lax.broadcasted_iota(jnp.int32, (N_DEV, n), 0)
        mine = slot == my_pos
        v = jnp.where(mine, jnp.float32(-jnp.inf), v)
        i = jnp.where(mine, BIG, i)
        vmax = jnp.maximum(jnp.max(v, axis=0), vmax_l)
        cand_r = jnp.min(jnp.where(v == vmax[None, :], i, BIG), axis=0)
        cand_l = jnp.where(vmax_l == vmax, gidx_l, BIG)
        out_ref[0, :] = vmax
        out_ref[1, :] = jnp.minimum(cand_r, cand_l)

    return pl.pallas_call(
        body,
        out_shape=jax.ShapeDtypeStruct((2, n), jnp.float32),
        in_specs=[pl.BlockSpec(memory_space=pltpu.VMEM)],
        out_specs=pl.BlockSpec(memory_space=pltpu.VMEM),
        scratch_shapes=[
            pltpu.VMEM((2, n), jnp.float32),
            pltpu.VMEM((N_DEV, 2, n), jnp.float32),
            pltpu.SemaphoreType.DMA((N_DEV - 1,)),
            pltpu.SemaphoreType.DMA((N_DEV,)),
        ],
        compiler_params=pltpu.CompilerParams(collective_id=0),
    )(x)
